# Initial kernel scaffold; baseline (speedup 1.0000x reference)
import jax
import jax.numpy as jnp
from jax.experimental import pallas as pl


def kernel(hidden_states, W_router, W_gate, W_up, W_down):
    raise NotImplementedError("write your pallas kernel here")


# speedup vs baseline: 2.6247x; 2.6247x over previous
import functools

import jax
import jax.numpy as jnp
from jax.experimental import pallas as pl
from jax.experimental.pallas import tpu as pltpu

NUM_EXPERTS = 8
TOP_K = 2
HIDDEN = 1024
INTERMEDIATE = 1024
NUM_TOKENS = 2048

TOK_TILE = 1024


def _router_weights(x, wr):
    logits = jnp.dot(x, wr, preferred_element_type=jnp.float32)
    p = jax.nn.softmax(logits, axis=-1)
    e_iota = jax.lax.broadcasted_iota(jnp.int32, p.shape, 1)
    m1 = jnp.max(logits, axis=-1, keepdims=True)
    i1 = jnp.min(jnp.where(logits == m1, e_iota, NUM_EXPERTS), axis=-1, keepdims=True)
    logits2 = jnp.where(e_iota == i1, -jnp.inf, logits)
    m2 = jnp.max(logits2, axis=-1, keepdims=True)
    i2 = jnp.min(jnp.where(logits2 == m2, e_iota, NUM_EXPERTS), axis=-1, keepdims=True)
    w1 = jnp.sum(jnp.where(e_iota == i1, p, 0.0), axis=-1, keepdims=True)
    w2 = jnp.sum(jnp.where(e_iota == i2, p, 0.0), axis=-1, keepdims=True)
    s = w1 + w2
    wd = jnp.where(e_iota == i1, w1 / s, 0.0) + jnp.where(e_iota == i2, w2 / s, 0.0)
    return wd


def _moe_body(x_ref, wr_ref, wg_ref, wu_ref, wd_ref, out_ref, wdense_ref):
    e = pl.program_id(1)

    @pl.when(e == 0)
    def _():
        wdense_ref[...] = _router_weights(x_ref[...], wr_ref[...])
        out_ref[...] = jnp.zeros_like(out_ref)

    x = x_ref[...]
    g = jnp.dot(x, wg_ref[0], preferred_element_type=jnp.float32)
    u = jnp.dot(x, wu_ref[0], preferred_element_type=jnp.float32)
    h = (g * jax.nn.sigmoid(g)) * u
    wd_all = wdense_ref[...]
    lane = jax.lax.broadcasted_iota(jnp.int32, wd_all.shape, 1)
    wcol = jnp.sum(jnp.where(lane == e, wd_all, 0.0), axis=1, keepdims=True)
    out_ref[...] += wcol * jnp.dot(h, wd_ref[0], preferred_element_type=jnp.float32)


def kernel(hidden_states, W_router, W_gate, W_up, W_down):
    n_tok_tiles = NUM_TOKENS // TOK_TILE
    grid = (n_tok_tiles, NUM_EXPERTS)
    out = pl.pallas_call(
        _moe_body,
        grid=grid,
        in_specs=[
            pl.BlockSpec((TOK_TILE, HIDDEN), lambda t, e: (t, 0)),
            pl.BlockSpec((HIDDEN, NUM_EXPERTS), lambda t, e: (0, 0)),
            pl.BlockSpec((1, HIDDEN, INTERMEDIATE), lambda t, e: (e, 0, 0)),
            pl.BlockSpec((1, HIDDEN, INTERMEDIATE), lambda t, e: (e, 0, 0)),
            pl.BlockSpec((1, INTERMEDIATE, HIDDEN), lambda t, e: (e, 0, 0)),
        ],
        out_specs=pl.BlockSpec((TOK_TILE, HIDDEN), lambda t, e: (t, 0)),
        out_shape=jax.ShapeDtypeStruct((NUM_TOKENS, HIDDEN), jnp.float32),
        scratch_shapes=[pltpu.VMEM((TOK_TILE, NUM_EXPERTS), jnp.float32)],
    )(hidden_states, W_router, W_gate, W_up, W_down)
    return out
